# baseline (speedup 1.0000x reference)
import functools

import jax
import jax.numpy as jnp
from jax import lax
from jax.experimental import pallas as pl
from jax.experimental.pallas import tpu as pltpu
from jax.experimental.pallas import tpu_sc as plsc

_T = 1000
_D = 128
_B = 16384

_NC = 2
_NS = 16
_NW = _NC * _NS
_BPW = _B // _NW


def _proj_silu_kernel(emb_ref, w_ref, b_ref, out_ref):
    x = lax.dot_general(
        emb_ref[...], w_ref[...],
        dimension_numbers=(((1,), (1,)), ((), ())),
        preferred_element_type=jnp.float32,
    ) + b_ref[...]
    out_ref[...] = x * jax.nn.sigmoid(x)


def _project_table(embedding, W1, b1):
    return pl.pallas_call(
        _proj_silu_kernel,
        out_shape=jax.ShapeDtypeStruct((_T, _D), jnp.float32),
    )(embedding, W1, b1.reshape(1, _D))


_vector_mesh = plsc.VectorSubcoreMesh(core_axis_name="c", subcore_axis_name="s")


@functools.partial(
    pl.kernel,
    mesh=_vector_mesh,
    out_type=jax.ShapeDtypeStruct((_B, _D), jnp.float32),
    scratch_types=[
        pltpu.VMEM((_BPW,), jnp.int32),
        pltpu.VMEM((_BPW, _D), jnp.float32),
        pltpu.SemaphoreType.DMA,
    ],
)
def _gather_kernel(table_hbm, idx_hbm, out_hbm, idx_v, rows_v, sem):
    wid = lax.axis_index("s") * _NC + lax.axis_index("c")
    base = wid * _BPW
    pltpu.sync_copy(idx_hbm.at[pl.ds(base, _BPW)], idx_v)
    pltpu.async_copy(table_hbm.at[idx_v], rows_v, sem).wait()
    pltpu.sync_copy(rows_v, out_hbm.at[pl.ds(base, _BPW)])


_UP_ROWS = 2048


def _copy_kernel(in_ref, out_ref):
    out_ref[...] = in_ref[...] * 1.0


def _tc_pass(x):
    return pl.pallas_call(
        _copy_kernel,
        grid=(_B // _UP_ROWS,),
        in_specs=[pl.BlockSpec((_UP_ROWS, _D), lambda i: (i, 0))],
        out_specs=pl.BlockSpec((_UP_ROWS, _D), lambda i: (i, 0)),
        out_shape=jax.ShapeDtypeStruct((_B, _D), jnp.float32),
    )(x)


def kernel(diffusion_step, embedding, W1, b1):
    table = _project_table(embedding, W1, b1)
    idx = jnp.asarray(diffusion_step, jnp.int32)
    return _tc_pass(_gather_kernel(table, idx))


# speedup vs baseline: 1.3548x; 1.3548x over previous
import functools

import jax
import jax.numpy as jnp
from jax import lax
from jax.experimental import pallas as pl
from jax.experimental.pallas import tpu as pltpu
from jax.experimental.pallas import tpu_sc as plsc

_T = 1000
_D = 128
_B = 16384

_NC = 2
_NS = 16
_NW = _NC * _NS
_BPW = _B // _NW


def _proj_silu_kernel(emb_ref, w_ref, b_ref, out_ref):
    x = lax.dot_general(
        emb_ref[...], w_ref[...],
        dimension_numbers=(((1,), (1,)), ((), ())),
        preferred_element_type=jnp.float32,
    ) + b_ref[...]
    out_ref[...] = x * jax.nn.sigmoid(x)


def _project_table(embedding, W1, b1):
    return pl.pallas_call(
        _proj_silu_kernel,
        out_shape=jax.ShapeDtypeStruct((_T, _D), jnp.float32),
    )(embedding, W1, b1.reshape(1, _D))


_vector_mesh = plsc.VectorSubcoreMesh(core_axis_name="c", subcore_axis_name="s")


@functools.partial(
    pl.kernel,
    mesh=_vector_mesh,
    out_type=jax.ShapeDtypeStruct((_B, _D), jnp.float32),
    scratch_types=[
        pltpu.VMEM((_BPW,), jnp.int32),
        pltpu.VMEM((_BPW, _D), jnp.float32),
        pltpu.SemaphoreType.DMA,
    ],
)
def _gather_kernel(table_hbm, idx_hbm, out_hbm, idx_v, rows_v, sem):
    wid = lax.axis_index("s") * _NC + lax.axis_index("c")
    base = wid * _BPW
    pltpu.sync_copy(idx_hbm.at[pl.ds(base, _BPW)], idx_v)
    pltpu.async_copy(table_hbm.at[idx_v], rows_v, sem).wait()
    pltpu.sync_copy(rows_v, out_hbm.at[pl.ds(base, _BPW)])


def kernel(diffusion_step, embedding, W1, b1):
    table = _project_table(embedding, W1, b1)
    idx = jnp.asarray(diffusion_step, jnp.int32)
    return _gather_kernel(table, idx)


# speedup vs baseline: 1.4082x; 1.0394x over previous
import functools

import jax
import jax.numpy as jnp
from jax import lax
from jax.experimental import pallas as pl
from jax.experimental.pallas import tpu as pltpu
from jax.experimental.pallas import tpu_sc as plsc

_T = 1000
_D = 128
_B = 16384

_NC = 2
_NS = 16
_NW = _NC * _NS
_BPW = _B // _NW

_CH = 128
_NCH = _BPW // _CH


def _proj_silu_kernel(emb_ref, w_ref, b_ref, out_ref):
    x = lax.dot_general(
        emb_ref[...], w_ref[...],
        dimension_numbers=(((1,), (1,)), ((), ())),
        preferred_element_type=jnp.float32,
    ) + b_ref[...]
    out_ref[...] = x * jax.nn.sigmoid(x)


def _project_table(embedding, W1, b1):
    return pl.pallas_call(
        _proj_silu_kernel,
        out_shape=jax.ShapeDtypeStruct((_T, _D), jnp.float32),
    )(embedding, W1, b1.reshape(1, _D))


_vector_mesh = plsc.VectorSubcoreMesh(core_axis_name="c", subcore_axis_name="s")


@functools.partial(
    pl.kernel,
    mesh=_vector_mesh,
    out_type=jax.ShapeDtypeStruct((_B, _D), jnp.float32),
    scratch_types=[
        pltpu.VMEM((_NCH, _CH), jnp.int32),
        pltpu.VMEM((_NCH, _CH, _D), jnp.float32),
        pltpu.VMEM_SHARED((_T, _D), jnp.float32),
        pltpu.SemaphoreType.DMA,
        pltpu.SemaphoreType.DMA,
        pltpu.SemaphoreType.DMA,
        pltpu.SemaphoreType.DMA,
        pltpu.SemaphoreType.DMA,
        pltpu.SemaphoreType.DMA,
        pltpu.SemaphoreType.DMA,
        pltpu.SemaphoreType.DMA,
    ],
)
def _gather_kernel(table_hbm, idx_hbm, out_hbm, idx_v, bufs, table_sp,
                   g0, g1, g2, g3, w0, w1, w2, w3):
    cid = lax.axis_index("c")
    sid = lax.axis_index("s")
    wid = sid * _NC + cid
    base = wid * _BPW

    @pl.when(sid < _NS - 1)
    def _():
        pltpu.sync_copy(table_hbm.at[pl.ds(sid * 64, 64)],
                        table_sp.at[pl.ds(sid * 64, 64)])

    @pl.when(sid == _NS - 1)
    def _():
        pltpu.sync_copy(table_hbm.at[pl.ds(960, _T - 960)],
                        table_sp.at[pl.ds(960, _T - 960)])

    for i in range(_NCH):
        pltpu.sync_copy(idx_hbm.at[pl.ds(base + i * _CH, _CH)], idx_v.at[i])

    plsc.subcore_barrier()

    gsems = (g0, g1, g2, g3)
    wsems = (w0, w1, w2, w3)
    gathers = [
        pltpu.async_copy(table_sp.at[idx_v.at[i]], bufs.at[i], gsems[i])
        for i in range(_NCH)
    ]
    writes = []
    for i in range(_NCH):
        gathers[i].wait()
        writes.append(
            pltpu.async_copy(bufs.at[i],
                             out_hbm.at[pl.ds(base + i * _CH, _CH)],
                             wsems[i]))
    for wr in writes:
        wr.wait()


def kernel(diffusion_step, embedding, W1, b1):
    table = _project_table(embedding, W1, b1)
    idx = jnp.asarray(diffusion_step, jnp.int32)
    return _gather_kernel(table, idx)
